# baseline (speedup 1.0000x reference)
import functools

import jax
import jax.numpy as jnp
from jax import lax
from jax.experimental import pallas as pl
from jax.experimental.pallas import tpu as pltpu
from jax.experimental.pallas import tpu_sc as plsc

N = 10000
E = 320000
D = 128
DA = 144
NC = 2
NS = 16
NW = NC * NS
EPT = E // NW
K = 80
NCHUNK = EPT // K
RPT = N // NS


def _scale_kernel(emb_ref, c_ref, out_ref):
    y = emb_ref[...]
    c = c_ref[0, 0]
    rc = jnp.sqrt(c)
    n = jnp.sqrt(jnp.sum(y * y, axis=1, keepdims=True))
    n = jnp.clip(n, 1e-10)
    z = rc * n
    atanh = 0.5 * jnp.log((1.0 + z) / (1.0 - z))
    t = (2.0 / rc) * atanh / n * y
    rows = y.shape[0]
    ones = jnp.ones((rows, 1), dtype=y.dtype)
    pad = jnp.zeros((rows, DA - D - 1), dtype=y.dtype)
    out_ref[...] = jnp.concatenate([t, ones, pad], axis=1)


def _seg_sum_kernel(taug, src_h, dst_h, zeros_h, out,
                    dst_all, src_a, src_b, rows_a, rows_b,
                    acc_sh, sem_a, sem_b, semi_a, semi_b, sem_sa, sem_sb):
    cid = lax.axis_index("c")
    sid = lax.axis_index("s")
    tid = cid * NS + sid

    pltpu.sync_copy(zeros_h, acc_sh.at[pl.ds(sid * RPT, RPT)])
    pltpu.sync_copy(dst_h.at[tid], dst_all)
    plsc.subcore_barrier()

    pltpu.sync_copy(src_h.at[tid, 0], src_a)
    pltpu.sync_copy(src_h.at[tid, 1], src_b)
    pltpu.async_copy(taug.at[src_a], rows_a, sem_a)
    pltpu.async_copy(taug.at[src_b], rows_b, sem_b)

    def body(i2, carry):
        ia = 2 * i2
        ib = ia + 1
        pltpu.make_async_copy(taug.at[src_a], rows_a, sem_a).wait()
        pltpu.async_copy(rows_a, acc_sh.at[dst_all.at[ia]], sem_sa, add=True)

        @pl.when(ia + 2 < NCHUNK)
        def _():
            pltpu.async_copy(src_h.at[tid, ia + 2], src_a, semi_a)

        pltpu.make_async_copy(taug.at[src_b], rows_b, sem_b).wait()
        pltpu.async_copy(rows_b, acc_sh.at[dst_all.at[ib]], sem_sb, add=True)

        @pl.when(ib + 2 < NCHUNK)
        def _():
            pltpu.async_copy(src_h.at[tid, ib + 2], src_b, semi_b)

        @pl.when(ia + 2 < NCHUNK)
        def _():
            pltpu.make_async_copy(rows_a, acc_sh.at[dst_all.at[ia]],
                                  sem_sa).wait()
            pltpu.make_async_copy(src_h.at[tid, ia + 2], src_a, semi_a).wait()
            pltpu.async_copy(taug.at[src_a], rows_a, sem_a)

        @pl.when(ib + 2 < NCHUNK)
        def _():
            pltpu.make_async_copy(rows_b, acc_sh.at[dst_all.at[ib]],
                                  sem_sb).wait()
            pltpu.make_async_copy(src_h.at[tid, ib + 2], src_b, semi_b).wait()
            pltpu.async_copy(taug.at[src_b], rows_b, sem_b)

        return carry

    lax.fori_loop(0, NCHUNK // 2, body, 0)
    pltpu.make_async_copy(taug.at[src_a], rows_a, sem_a).wait()
    pltpu.sync_copy(rows_a, acc_sh.at[dst_all.at[NCHUNK - 1]], add=True)
    pltpu.make_async_copy(rows_b, acc_sh.at[dst_all.at[NCHUNK - 2]],
                          sem_sb).wait()

    plsc.subcore_barrier()
    pltpu.sync_copy(acc_sh.at[pl.ds(sid * RPT, RPT)],
                    out.at[cid, pl.ds(sid * RPT, RPT)])


def _finish_kernel(part_ref, w_ref, c_ref, out_ref):
    p = part_ref[...]
    s = p[0] + p[1]
    c = c_ref[0, 0]
    rc = jnp.sqrt(c)
    agg = s[:, :D]
    cnt = jnp.clip(s[:, D:D + 1], 1.0)
    neigh = jnp.dot(agg, w_ref[...], preferred_element_type=jnp.float32) / cnt
    m = jnp.sqrt(jnp.sum(neigh * neigh, axis=1, keepdims=True))
    m = jnp.clip(m, 1e-10)
    out_ref[...] = jnp.tanh(rc * m * 0.5) * neigh / (rc * m)


def kernel(edge_index, node_embeddings, gc_weight, curvature):
    c2d = curvature.reshape(1, 1).astype(jnp.float32)
    src = edge_index[0].astype(jnp.int32).reshape(NW, NCHUNK, K)
    dst = edge_index[1].astype(jnp.int32).reshape(NW, NCHUNK, K)

    rows_blk = 1000
    taug = pl.pallas_call(
        _scale_kernel,
        grid=(N // rows_blk,),
        in_specs=[
            pl.BlockSpec((rows_blk, D), lambda i: (i, 0)),
            pl.BlockSpec(memory_space=pltpu.SMEM),
        ],
        out_specs=pl.BlockSpec((rows_blk, DA), lambda i: (i, 0)),
        out_shape=jax.ShapeDtypeStruct((N, DA), jnp.float32),
    )(node_embeddings, c2d)

    zeros_h = jnp.zeros((RPT, DA), dtype=jnp.float32)

    mesh = plsc.VectorSubcoreMesh(core_axis_name="c", subcore_axis_name="s",
                                  num_cores=NC, num_subcores=NS)
    partials = pl.kernel(
        _seg_sum_kernel,
        out_type=jax.ShapeDtypeStruct((NC, N, DA), jnp.float32),
        mesh=mesh,
        scratch_types=[
            pltpu.VMEM((NCHUNK, K), jnp.int32),
            pltpu.VMEM((K,), jnp.int32),
            pltpu.VMEM((K,), jnp.int32),
            pltpu.VMEM((K, DA), jnp.float32),
            pltpu.VMEM((K, DA), jnp.float32),
            pltpu.VMEM_SHARED((N, DA), jnp.float32),
            pltpu.SemaphoreType.DMA,
            pltpu.SemaphoreType.DMA,
            pltpu.SemaphoreType.DMA,
            pltpu.SemaphoreType.DMA,
            pltpu.SemaphoreType.DMA,
            pltpu.SemaphoreType.DMA,
        ],
        compiler_params=pltpu.CompilerParams(use_tc_tiling_on_sc=False),
    )(taug, src, dst, zeros_h)

    out = pl.pallas_call(
        _finish_kernel,
        grid=(N // rows_blk,),
        in_specs=[
            pl.BlockSpec((NC, rows_blk, DA), lambda i: (0, i, 0)),
            pl.BlockSpec((D, D), lambda i: (0, 0)),
            pl.BlockSpec(memory_space=pltpu.SMEM),
        ],
        out_specs=pl.BlockSpec((rows_blk, D), lambda i: (i, 0)),
        out_shape=jax.ShapeDtypeStruct((N, D), jnp.float32),
    )(partials, gc_weight, c2d)

    return out


# speedup vs baseline: 1.0802x; 1.0802x over previous
import jax
import jax.numpy as jnp
from jax import lax
from jax.experimental import pallas as pl
from jax.experimental.pallas import tpu as pltpu
from jax.experimental.pallas import tpu_sc as plsc

N = 10000
E = 320000
D = 128
DA = 144
NC = 2
NS = 16
NW = NC * NS
EPT = E // NW
K = 40
NCHUNK = EPT // K
S = 5
GLEAD = 2
RPT = N // NS


def _scale_kernel(emb_ref, c_ref, out_ref):
    y = emb_ref[...]
    c = c_ref[0, 0]
    rc = jnp.sqrt(c)
    n = jnp.sqrt(jnp.sum(y * y, axis=1, keepdims=True))
    n = jnp.clip(n, 1e-10)
    z = rc * n
    atanh = 0.5 * jnp.log((1.0 + z) / (1.0 - z))
    t = (2.0 / rc) * atanh / n * y
    rows = y.shape[0]
    ones = jnp.ones((rows, 1), dtype=y.dtype)
    pad = jnp.zeros((rows, DA - D - 1), dtype=y.dtype)
    out_ref[...] = jnp.concatenate([t, ones, pad], axis=1)


def _seg_sum_kernel(taug, src_h, dst_h, zeros_h, out,
                    rows, srcb, dst_all, acc_sh, semg, sems, semi):
    cid = lax.axis_index("c")
    sid = lax.axis_index("s")
    tid = cid * NS + sid

    pltpu.sync_copy(zeros_h, acc_sh.at[pl.ds(sid * RPT, RPT)])
    pltpu.sync_copy(dst_h.at[tid], dst_all)
    plsc.subcore_barrier()

    def fire_gather(c, b):
        pltpu.async_copy(taug.at[srcb[b]], rows[b], semg[b])

    def wait_gather(b):
        pltpu.make_async_copy(taug.at[srcb[b]], rows[b], semg[b]).wait()

    def fire_scatter(c, b):
        pltpu.async_copy(rows[b], acc_sh.at[dst_all.at[c]], sems[b], add=True)

    def wait_scatter(c, b):
        pltpu.make_async_copy(rows[b], acc_sh.at[dst_all.at[c]],
                              sems[b]).wait()

    def fire_idx(c, b):
        pltpu.async_copy(src_h.at[tid, c], srcb[b], semi[b])

    def wait_idx(b):
        pltpu.make_async_copy(src_h.at[tid, 0], srcb[b], semi[b]).wait()

    for b in range(S):
        pltpu.sync_copy(src_h.at[tid, b], srcb[b])
    fire_gather(0, 0)
    fire_gather(1, 1)

    def turn(c, b, *, sync_idx=False, no_idx=False, no_gather=False):
        wait_gather(b)
        fire_scatter(c, b)
        if not no_idx:
            fire_idx(c + S, b)
        if not no_gather:
            b2 = (b + GLEAD) % S
            if not sync_idx:
                wait_scatter(c - (S - GLEAD), b2)
                wait_idx(b2)
            fire_gather(c + GLEAD, b2)

    turn(0, 0, sync_idx=True)
    turn(1, 1, sync_idx=True)
    turn(2, 2, sync_idx=True)
    turn(3, 3)
    turn(4, 4)

    def body(g, carry):
        for b in range(S):
            c = g * S + b
            turn(c, b)
        return carry

    lax.fori_loop(1, NCHUNK // S - 1, body, 0)

    turn(NCHUNK - S + 0, 0, no_idx=True)
    turn(NCHUNK - S + 1, 1, no_idx=True)
    turn(NCHUNK - S + 2, 2, no_idx=True)
    turn(NCHUNK - S + 3, 3, no_idx=True, no_gather=True)
    turn(NCHUNK - S + 4, 4, no_idx=True, no_gather=True)
    for b in range(S):
        wait_scatter(NCHUNK - S + b, b)

    plsc.subcore_barrier()
    pltpu.sync_copy(acc_sh.at[pl.ds(sid * RPT, RPT)],
                    out.at[cid, pl.ds(sid * RPT, RPT)])


def _finish_kernel(part_ref, w_ref, c_ref, out_ref):
    p = part_ref[...]
    s = p[0] + p[1]
    c = c_ref[0, 0]
    rc = jnp.sqrt(c)
    agg = s[:, :D]
    cnt = jnp.clip(s[:, D:D + 1], 1.0)
    neigh = jnp.dot(agg, w_ref[...], preferred_element_type=jnp.float32) / cnt
    m = jnp.sqrt(jnp.sum(neigh * neigh, axis=1, keepdims=True))
    m = jnp.clip(m, 1e-10)
    out_ref[...] = jnp.tanh(rc * m * 0.5) * neigh / (rc * m)


def kernel(edge_index, node_embeddings, gc_weight, curvature):
    c2d = curvature.reshape(1, 1).astype(jnp.float32)
    src = edge_index[0].astype(jnp.int32).reshape(NW, NCHUNK, K)
    dst = edge_index[1].astype(jnp.int32).reshape(NW, NCHUNK, K)

    rows_blk = 1000
    taug = pl.pallas_call(
        _scale_kernel,
        grid=(N // rows_blk,),
        in_specs=[
            pl.BlockSpec((rows_blk, D), lambda i: (i, 0)),
            pl.BlockSpec(memory_space=pltpu.SMEM),
        ],
        out_specs=pl.BlockSpec((rows_blk, DA), lambda i: (i, 0)),
        out_shape=jax.ShapeDtypeStruct((N, DA), jnp.float32),
    )(node_embeddings, c2d)

    zeros_h = jnp.zeros((RPT, DA), dtype=jnp.float32)

    mesh = plsc.VectorSubcoreMesh(core_axis_name="c", subcore_axis_name="s",
                                  num_cores=NC, num_subcores=NS)
    partials = pl.kernel(
        _seg_sum_kernel,
        out_type=jax.ShapeDtypeStruct((NC, N, DA), jnp.float32),
        mesh=mesh,
        scratch_types=[
            [pltpu.VMEM((K, DA), jnp.float32) for _ in range(S)],
            [pltpu.VMEM((K,), jnp.int32) for _ in range(S)],
            pltpu.VMEM((NCHUNK, K), jnp.int32),
            pltpu.VMEM_SHARED((N, DA), jnp.float32),
            [pltpu.SemaphoreType.DMA for _ in range(S)],
            [pltpu.SemaphoreType.DMA for _ in range(S)],
            [pltpu.SemaphoreType.DMA for _ in range(S)],
        ],
        compiler_params=pltpu.CompilerParams(use_tc_tiling_on_sc=False),
    )(taug, src, dst, zeros_h)

    out = pl.pallas_call(
        _finish_kernel,
        grid=(N // rows_blk,),
        in_specs=[
            pl.BlockSpec((NC, rows_blk, DA), lambda i: (0, i, 0)),
            pl.BlockSpec((D, D), lambda i: (0, 0)),
            pl.BlockSpec(memory_space=pltpu.SMEM),
        ],
        out_specs=pl.BlockSpec((rows_blk, D), lambda i: (i, 0)),
        out_shape=jax.ShapeDtypeStruct((N, D), jnp.float32),
    )(partials, gc_weight, c2d)

    return out


# speedup vs baseline: 1.2649x; 1.1710x over previous
import jax
import jax.numpy as jnp
from jax import lax
from jax.experimental import pallas as pl
from jax.experimental.pallas import tpu as pltpu
from jax.experimental.pallas import tpu_sc as plsc

N = 10000
E = 320000
D = 128
DA = 144
NC = 2
NS = 16
NW = NC * NS
EPT = E // NW
K = 40
NCHUNK = EPT // K
S = 5
GLEAD = 2
RPT = N // NS


def _scale_kernel(emb_ref, c_ref, out_ref):
    y = emb_ref[...]
    c = c_ref[0, 0]
    rc = jnp.sqrt(c)
    n = jnp.sqrt(jnp.sum(y * y, axis=1, keepdims=True))
    n = jnp.clip(n, 1e-10)
    z = rc * n
    atanh = 0.5 * jnp.log((1.0 + z) / (1.0 - z))
    t = (2.0 / rc) * atanh / n * y
    out_ref[...] = t


def _seg_sum_kernel(taug, src_h, dst_h, zeros_h, zc_h, out, out_cnt,
                    rows, srcb, dst_all, cnt_t, acc_sh, semg, sems, semi):
    cid = lax.axis_index("c")
    sid = lax.axis_index("s")
    tid = cid * NS + sid

    pltpu.sync_copy(zeros_h, acc_sh.at[pl.ds(sid * RPT, RPT)])
    pltpu.sync_copy(dst_h.at[tid], dst_all)
    pltpu.sync_copy(zc_h, cnt_t)
    plsc.subcore_barrier()

    def fire_gather(c, b):
        pltpu.async_copy(taug.at[srcb[b]], rows[b], semg[b])

    def wait_gather(b):
        pltpu.make_async_copy(taug.at[srcb[b]], rows[b], semg[b]).wait()

    ones16 = jnp.ones((16,), jnp.float32)
    tailmask = lax.iota(jnp.int32, 16) >= 8

    def count_chunk(c):
        v0 = dst_all[c, pl.ds(0, 16)]
        plsc.addupdate_scatter(cnt_t, [v0], ones16)
        v1 = dst_all[c, pl.ds(16, 16)]
        plsc.addupdate_scatter(cnt_t, [v1], ones16)
        v2 = dst_all[c, pl.ds(24, 16)]
        plsc.addupdate_scatter(cnt_t, [v2], ones16, mask=tailmask)

    def fire_scatter(c, b):
        pltpu.async_copy(rows[b], acc_sh.at[dst_all.at[c]], sems[b], add=True)

    def wait_scatter(c, b):
        pltpu.make_async_copy(rows[b], acc_sh.at[dst_all.at[c]],
                              sems[b]).wait()

    def fire_idx(c, b):
        pltpu.async_copy(src_h.at[tid, c], srcb[b], semi[b])

    def wait_idx(b):
        pltpu.make_async_copy(src_h.at[tid, 0], srcb[b], semi[b]).wait()

    for b in range(S):
        pltpu.sync_copy(src_h.at[tid, b], srcb[b])
    fire_gather(0, 0)
    fire_gather(1, 1)

    def turn(c, b, *, sync_idx=False, no_idx=False, no_gather=False):
        count_chunk(c)
        wait_gather(b)
        fire_scatter(c, b)
        if not no_idx:
            fire_idx(c + S, b)
        if not no_gather:
            b2 = (b + GLEAD) % S
            if not sync_idx:
                wait_scatter(c - (S - GLEAD), b2)
                wait_idx(b2)
            fire_gather(c + GLEAD, b2)

    turn(0, 0, sync_idx=True)
    turn(1, 1, sync_idx=True)
    turn(2, 2, sync_idx=True)
    turn(3, 3)
    turn(4, 4)

    def body(g, carry):
        for b in range(S):
            c = g * S + b
            turn(c, b)
        return carry

    lax.fori_loop(1, NCHUNK // S - 1, body, 0)

    turn(NCHUNK - S + 0, 0, no_idx=True)
    turn(NCHUNK - S + 1, 1, no_idx=True)
    turn(NCHUNK - S + 2, 2, no_idx=True)
    turn(NCHUNK - S + 3, 3, no_idx=True, no_gather=True)
    turn(NCHUNK - S + 4, 4, no_idx=True, no_gather=True)
    for b in range(S):
        wait_scatter(NCHUNK - S + b, b)

    pltpu.sync_copy(cnt_t, out_cnt.at[tid])
    plsc.subcore_barrier()
    pltpu.sync_copy(acc_sh.at[pl.ds(sid * RPT, RPT)],
                    out.at[cid, pl.ds(sid * RPT, RPT)])


def _finish_kernel(part_ref, cnt_ref, w_ref, c_ref, out_ref):
    p = part_ref[...]
    agg = p[0] + p[1]
    c = c_ref[0, 0]
    rc = jnp.sqrt(c)
    cnt = jnp.clip(jnp.sum(cnt_ref[...], axis=1, keepdims=True), 1.0)
    neigh = jnp.dot(agg, w_ref[...], preferred_element_type=jnp.float32) / cnt
    m = jnp.sqrt(jnp.sum(neigh * neigh, axis=1, keepdims=True))
    m = jnp.clip(m, 1e-10)
    out_ref[...] = jnp.tanh(rc * m * 0.5) * neigh / (rc * m)


def kernel(edge_index, node_embeddings, gc_weight, curvature):
    c2d = curvature.reshape(1, 1).astype(jnp.float32)
    src = edge_index[0].astype(jnp.int32).reshape(NW, NCHUNK, K)
    dst = edge_index[1].astype(jnp.int32).reshape(NW, NCHUNK, K)

    rows_blk = 1000
    taug = pl.pallas_call(
        _scale_kernel,
        grid=(N // rows_blk,),
        in_specs=[
            pl.BlockSpec((rows_blk, D), lambda i: (i, 0)),
            pl.BlockSpec(memory_space=pltpu.SMEM),
        ],
        out_specs=pl.BlockSpec((rows_blk, D), lambda i: (i, 0)),
        out_shape=jax.ShapeDtypeStruct((N, D), jnp.float32),
    )(node_embeddings, c2d)

    zeros_h = jnp.zeros((RPT, D), dtype=jnp.float32)
    zc_h = jnp.zeros((N,), dtype=jnp.float32)

    mesh = plsc.VectorSubcoreMesh(core_axis_name="c", subcore_axis_name="s",
                                  num_cores=NC, num_subcores=NS)
    partials, counts = pl.kernel(
        _seg_sum_kernel,
        out_type=[jax.ShapeDtypeStruct((NC, N, D), jnp.float32),
                  jax.ShapeDtypeStruct((NW, N), jnp.float32)],
        mesh=mesh,
        scratch_types=[
            [pltpu.VMEM((K, D), jnp.float32) for _ in range(S)],
            [pltpu.VMEM((K,), jnp.int32) for _ in range(S)],
            pltpu.VMEM((NCHUNK, K), jnp.int32),
            pltpu.VMEM((N,), jnp.float32),
            pltpu.VMEM_SHARED((N, D), jnp.float32),
            [pltpu.SemaphoreType.DMA for _ in range(S)],
            [pltpu.SemaphoreType.DMA for _ in range(S)],
            [pltpu.SemaphoreType.DMA for _ in range(S)],
        ],
        compiler_params=pltpu.CompilerParams(use_tc_tiling_on_sc=False,
                                             needs_layout_passes=False),
    )(taug, src, dst, zeros_h, zc_h)

    out = pl.pallas_call(
        _finish_kernel,
        grid=(N // rows_blk,),
        in_specs=[
            pl.BlockSpec((NC, rows_blk, D), lambda i: (0, i, 0)),
            pl.BlockSpec((rows_blk, NW), lambda i: (i, 0)),
            pl.BlockSpec((D, D), lambda i: (0, 0)),
            pl.BlockSpec(memory_space=pltpu.SMEM),
        ],
        out_specs=pl.BlockSpec((rows_blk, D), lambda i: (i, 0)),
        out_shape=jax.ShapeDtypeStruct((N, D), jnp.float32),
    )(partials, counts.T, gc_weight, c2d)

    return out


# speedup vs baseline: 1.4894x; 1.1775x over previous
import jax
import jax.numpy as jnp
from jax import lax
from jax.experimental import pallas as pl
from jax.experimental.pallas import tpu as pltpu
from jax.experimental.pallas import tpu_sc as plsc

N = 10000
E = 320000
D = 128
DA = 144
NC = 2
NS = 16
NW = NC * NS
EPT = E // NW
K = 40
NCHUNK = EPT // K
S = 5
GLEAD = 2
RPT = N // NS


def _scale_kernel(emb_ref, c_ref, out_ref):
    y = emb_ref[...]
    c = c_ref[0, 0]
    rc = jnp.sqrt(c)
    n = jnp.sqrt(jnp.sum(y * y, axis=1, keepdims=True))
    n = jnp.clip(n, 1e-10)
    z = rc * n
    atanh = 0.5 * jnp.log((1.0 + z) / (1.0 - z))
    t = (2.0 / rc) * atanh / n * y
    out_ref[...] = t


def _seg_sum_kernel(taug, src_h, dst_h, zeros_h, zc_h, out, out_cnt,
                    rows, srcb, dst_all, cnt_t, acc_sh, semg, sems, semi):
    cid = lax.axis_index("c")
    sid = lax.axis_index("s")
    tid = cid * NS + sid

    pltpu.sync_copy(zeros_h, acc_sh.at[pl.ds(sid * RPT, RPT)])
    pltpu.sync_copy(dst_h.at[tid], dst_all)
    pltpu.sync_copy(zc_h, cnt_t)
    plsc.subcore_barrier()

    def fire_gather(c, b):
        pltpu.async_copy(taug.at[srcb[b]], rows[b], semg[b])

    def wait_gather(b):
        pltpu.make_async_copy(taug.at[srcb[b]], rows[b], semg[b]).wait()

    ones16 = jnp.ones((16,), jnp.float32)
    tailmask = lax.iota(jnp.int32, 16) >= 8

    def count_chunk(c):
        v0 = dst_all[c, pl.ds(0, 16)]
        plsc.addupdate_scatter(cnt_t, [v0], ones16)
        v1 = dst_all[c, pl.ds(16, 16)]
        plsc.addupdate_scatter(cnt_t, [v1], ones16)
        v2 = dst_all[c, pl.ds(24, 16)]
        plsc.addupdate_scatter(cnt_t, [v2], ones16, mask=tailmask)

    def fire_scatter(c, b):
        pltpu.async_copy(rows[b], acc_sh.at[dst_all.at[c]], sems[b], add=True)

    def wait_scatter(c, b):
        pltpu.make_async_copy(rows[b], acc_sh.at[dst_all.at[c]],
                              sems[b]).wait()

    def fire_idx(c, b):
        pltpu.async_copy(src_h.at[tid, c], srcb[b], semi[b])

    def wait_idx(b):
        pltpu.make_async_copy(src_h.at[tid, 0], srcb[b], semi[b]).wait()

    for b in range(S):
        pltpu.sync_copy(src_h.at[tid, b], srcb[b])
    fire_gather(0, 0)
    fire_gather(1, 1)

    def turn(c, b, *, sync_idx=False, no_idx=False, no_gather=False):
        count_chunk(c)
        if not no_gather:
            b2 = (b + GLEAD) % S
            if not sync_idx:
                wait_scatter(c - (S - GLEAD), b2)
                wait_idx(b2)
            fire_gather(c + GLEAD, b2)
        wait_gather(b)
        fire_scatter(c, b)
        if not no_idx:
            fire_idx(c + S, b)

    turn(0, 0, sync_idx=True)
    turn(1, 1, sync_idx=True)
    turn(2, 2, sync_idx=True)
    turn(3, 3)
    turn(4, 4)

    def body(g, carry):
        for b in range(S):
            c = g * S + b
            turn(c, b)
        return carry

    lax.fori_loop(1, NCHUNK // S - 1, body, 0)

    turn(NCHUNK - S + 0, 0, no_idx=True)
    turn(NCHUNK - S + 1, 1, no_idx=True)
    turn(NCHUNK - S + 2, 2, no_idx=True)
    turn(NCHUNK - S + 3, 3, no_idx=True, no_gather=True)
    turn(NCHUNK - S + 4, 4, no_idx=True, no_gather=True)
    for b in range(S):
        wait_scatter(NCHUNK - S + b, b)

    pltpu.sync_copy(cnt_t, out_cnt.at[tid])
    plsc.subcore_barrier()
    pltpu.sync_copy(acc_sh.at[pl.ds(sid * RPT, RPT)],
                    out.at[cid, pl.ds(sid * RPT, RPT)])


def _finish_kernel(part_ref, cnt_ref, w_ref, c_ref, out_ref):
    p = part_ref[...]
    agg = p[0] + p[1]
    c = c_ref[0, 0]
    rc = jnp.sqrt(c)
    cnt = jnp.clip(jnp.sum(cnt_ref[...], axis=1, keepdims=True), 1.0)
    neigh = jnp.dot(agg, w_ref[...], preferred_element_type=jnp.float32) / cnt
    m = jnp.sqrt(jnp.sum(neigh * neigh, axis=1, keepdims=True))
    m = jnp.clip(m, 1e-10)
    out_ref[...] = jnp.tanh(rc * m * 0.5) * neigh / (rc * m)


def kernel(edge_index, node_embeddings, gc_weight, curvature):
    c2d = curvature.reshape(1, 1).astype(jnp.float32)
    src = edge_index[0].astype(jnp.int32).reshape(NW, NCHUNK, K)
    dst = edge_index[1].astype(jnp.int32).reshape(NW, NCHUNK, K)

    rows_blk = 1000
    taug = pl.pallas_call(
        _scale_kernel,
        grid=(N // rows_blk,),
        in_specs=[
            pl.BlockSpec((rows_blk, D), lambda i: (i, 0)),
            pl.BlockSpec(memory_space=pltpu.SMEM),
        ],
        out_specs=pl.BlockSpec((rows_blk, D), lambda i: (i, 0)),
        out_shape=jax.ShapeDtypeStruct((N, D), jnp.float32),
    )(node_embeddings, c2d)

    zeros_h = jnp.zeros((RPT, D), dtype=jnp.float32)
    zc_h = jnp.zeros((N,), dtype=jnp.float32)

    mesh = plsc.VectorSubcoreMesh(core_axis_name="c", subcore_axis_name="s",
                                  num_cores=NC, num_subcores=NS)
    partials, counts = pl.kernel(
        _seg_sum_kernel,
        out_type=[jax.ShapeDtypeStruct((NC, N, D), jnp.float32),
                  jax.ShapeDtypeStruct((NW, N), jnp.float32)],
        mesh=mesh,
        scratch_types=[
            [pltpu.VMEM((K, D), jnp.float32) for _ in range(S)],
            [pltpu.VMEM((K,), jnp.int32) for _ in range(S)],
            pltpu.VMEM((NCHUNK, K), jnp.int32),
            pltpu.VMEM((N,), jnp.float32),
            pltpu.VMEM_SHARED((N, D), jnp.float32),
            [pltpu.SemaphoreType.DMA for _ in range(S)],
            [pltpu.SemaphoreType.DMA for _ in range(S)],
            [pltpu.SemaphoreType.DMA for _ in range(S)],
        ],
        compiler_params=pltpu.CompilerParams(use_tc_tiling_on_sc=False,
                                             needs_layout_passes=False),
    )(taug, src, dst, zeros_h, zc_h)

    out = pl.pallas_call(
        _finish_kernel,
        grid=(N // rows_blk,),
        in_specs=[
            pl.BlockSpec((NC, rows_blk, D), lambda i: (0, i, 0)),
            pl.BlockSpec((rows_blk, NW), lambda i: (i, 0)),
            pl.BlockSpec((D, D), lambda i: (0, 0)),
            pl.BlockSpec(memory_space=pltpu.SMEM),
        ],
        out_specs=pl.BlockSpec((rows_blk, D), lambda i: (i, 0)),
        out_shape=jax.ShapeDtypeStruct((N, D), jnp.float32),
    )(partials, counts.T, gc_weight, c2d)

    return out


# speedup vs baseline: 1.5559x; 1.0446x over previous
import jax
import jax.numpy as jnp
from jax import lax
from jax.experimental import pallas as pl
from jax.experimental.pallas import tpu as pltpu
from jax.experimental.pallas import tpu_sc as plsc

N = 10000
E = 320000
D = 128
DA = 144
NC = 2
NS = 16
NW = NC * NS
EPT = E // NW
K = 40
NCHUNK = EPT // K
S = 5
GLEAD = 3
RPT = N // NS


def _scale_kernel(emb_ref, c_ref, out_ref):
    y = emb_ref[...]
    c = c_ref[0, 0]
    rc = jnp.sqrt(c)
    n = jnp.sqrt(jnp.sum(y * y, axis=1, keepdims=True))
    n = jnp.clip(n, 1e-10)
    z = rc * n
    atanh = 0.5 * jnp.log((1.0 + z) / (1.0 - z))
    t = (2.0 / rc) * atanh / n * y
    out_ref[...] = t


def _seg_sum_kernel(taug, src_h, dst_h, zeros_h, zc_h, out, out_cnt,
                    rows, srcb, dst_all, cnt_t, acc_sh, semg, sems, semi):
    cid = lax.axis_index("c")
    sid = lax.axis_index("s")
    tid = cid * NS + sid

    pltpu.sync_copy(zeros_h, acc_sh.at[pl.ds(sid * RPT, RPT)])
    pltpu.sync_copy(dst_h.at[tid], dst_all)
    pltpu.sync_copy(zc_h, cnt_t)
    plsc.subcore_barrier()

    def fire_gather(c, b):
        pltpu.async_copy(taug.at[srcb[b]], rows[b], semg[b])

    def wait_gather(b):
        pltpu.make_async_copy(taug.at[srcb[b]], rows[b], semg[b]).wait()

    ones16 = jnp.ones((16,), jnp.float32)
    tailmask = lax.iota(jnp.int32, 16) >= 8

    def count_chunk(c):
        v0 = dst_all[c, pl.ds(0, 16)]
        plsc.addupdate_scatter(cnt_t, [v0], ones16)
        v1 = dst_all[c, pl.ds(16, 16)]
        plsc.addupdate_scatter(cnt_t, [v1], ones16)
        v2 = dst_all[c, pl.ds(24, 16)]
        plsc.addupdate_scatter(cnt_t, [v2], ones16, mask=tailmask)

    def fire_scatter(c, b):
        pltpu.async_copy(rows[b], acc_sh.at[dst_all.at[c]], sems[b], add=True)

    def wait_scatter(c, b):
        pltpu.make_async_copy(rows[b], acc_sh.at[dst_all.at[c]],
                              sems[b]).wait()

    def fire_idx(c, b):
        pltpu.async_copy(src_h.at[tid, c], srcb[b], semi[b])

    def wait_idx(b):
        pltpu.make_async_copy(src_h.at[tid, 0], srcb[b], semi[b]).wait()

    for b in range(S):
        pltpu.sync_copy(src_h.at[tid, b], srcb[b])
    for b in range(GLEAD):
        fire_gather(b, b)

    def turn(c, b, *, sync_idx=False, no_idx=False, no_gather=False):
        count_chunk(c)
        if not no_gather:
            b2 = (b + GLEAD) % S
            if not sync_idx:
                wait_scatter(c - (S - GLEAD), b2)
                wait_idx(b2)
            fire_gather(c + GLEAD, b2)
        wait_gather(b)
        fire_scatter(c, b)
        if not no_idx:
            fire_idx(c + S, b)

    for b in range(S):
        turn(b, b, sync_idx=(b < S - GLEAD))

    def body(g, carry):
        for b in range(S):
            c = g * S + b
            turn(c, b)
        return carry

    lax.fori_loop(1, NCHUNK // S - 1, body, 0)

    for b in range(S):
        turn(NCHUNK - S + b, b, no_idx=True, no_gather=(b >= S - GLEAD))
    for b in range(S):
        wait_scatter(NCHUNK - S + b, b)

    pltpu.sync_copy(cnt_t, out_cnt.at[tid])
    plsc.subcore_barrier()
    pltpu.sync_copy(acc_sh.at[pl.ds(sid * RPT, RPT)],
                    out.at[cid, pl.ds(sid * RPT, RPT)])


def _finish_kernel(part_ref, cnt_ref, w_ref, c_ref, out_ref):
    p = part_ref[...]
    agg = p[0] + p[1]
    c = c_ref[0, 0]
    rc = jnp.sqrt(c)
    cnt = jnp.clip(jnp.sum(cnt_ref[...], axis=1, keepdims=True), 1.0)
    neigh = jnp.dot(agg, w_ref[...], preferred_element_type=jnp.float32) / cnt
    m = jnp.sqrt(jnp.sum(neigh * neigh, axis=1, keepdims=True))
    m = jnp.clip(m, 1e-10)
    out_ref[...] = jnp.tanh(rc * m * 0.5) * neigh / (rc * m)


def kernel(edge_index, node_embeddings, gc_weight, curvature):
    c2d = curvature.reshape(1, 1).astype(jnp.float32)
    src = edge_index[0].astype(jnp.int32).reshape(NW, NCHUNK, K)
    dst = edge_index[1].astype(jnp.int32).reshape(NW, NCHUNK, K)

    rows_blk = 1000
    taug = pl.pallas_call(
        _scale_kernel,
        grid=(N // rows_blk,),
        in_specs=[
            pl.BlockSpec((rows_blk, D), lambda i: (i, 0)),
            pl.BlockSpec(memory_space=pltpu.SMEM),
        ],
        out_specs=pl.BlockSpec((rows_blk, D), lambda i: (i, 0)),
        out_shape=jax.ShapeDtypeStruct((N, D), jnp.float32),
    )(node_embeddings, c2d)

    zeros_h = jnp.zeros((RPT, D), dtype=jnp.float32)
    zc_h = jnp.zeros((N,), dtype=jnp.float32)

    mesh = plsc.VectorSubcoreMesh(core_axis_name="c", subcore_axis_name="s",
                                  num_cores=NC, num_subcores=NS)
    partials, counts = pl.kernel(
        _seg_sum_kernel,
        out_type=[jax.ShapeDtypeStruct((NC, N, D), jnp.float32),
                  jax.ShapeDtypeStruct((NW, N), jnp.float32)],
        mesh=mesh,
        scratch_types=[
            [pltpu.VMEM((K, D), jnp.float32) for _ in range(S)],
            [pltpu.VMEM((K,), jnp.int32) for _ in range(S)],
            pltpu.VMEM((NCHUNK, K), jnp.int32),
            pltpu.VMEM((N,), jnp.float32),
            pltpu.VMEM_SHARED((N, D), jnp.float32),
            [pltpu.SemaphoreType.DMA for _ in range(S)],
            [pltpu.SemaphoreType.DMA for _ in range(S)],
            [pltpu.SemaphoreType.DMA for _ in range(S)],
        ],
        compiler_params=pltpu.CompilerParams(use_tc_tiling_on_sc=False,
                                             needs_layout_passes=False),
    )(taug, src, dst, zeros_h, zc_h)

    out = pl.pallas_call(
        _finish_kernel,
        grid=(N // rows_blk,),
        in_specs=[
            pl.BlockSpec((NC, rows_blk, D), lambda i: (0, i, 0)),
            pl.BlockSpec((rows_blk, NW), lambda i: (i, 0)),
            pl.BlockSpec((D, D), lambda i: (0, 0)),
            pl.BlockSpec(memory_space=pltpu.SMEM),
        ],
        out_specs=pl.BlockSpec((rows_blk, D), lambda i: (i, 0)),
        out_shape=jax.ShapeDtypeStruct((N, D), jnp.float32),
    )(partials, counts.T, gc_weight, c2d)

    return out


# speedup vs baseline: 1.6972x; 1.0908x over previous
import jax
import jax.numpy as jnp
from jax import lax
from jax.experimental import pallas as pl
from jax.experimental.pallas import tpu as pltpu
from jax.experimental.pallas import tpu_sc as plsc

N = 10000
E = 320000
D = 128
DA = 144
NC = 2
NS = 16
NW = NC * NS
EPT = E // NW
K = 40
NCHUNK = EPT // K
S = 5
GLEAD = 3
RPT = N // NS


def _scale_kernel(emb_ref, c_ref, out_ref):
    y = emb_ref[...]
    c = c_ref[0, 0]
    rc = jnp.sqrt(c)
    n = jnp.sqrt(jnp.sum(y * y, axis=1, keepdims=True))
    n = jnp.clip(n, 1e-10)
    z = rc * n
    atanh = 0.5 * jnp.log((1.0 + z) / (1.0 - z))
    t = (2.0 / rc) * atanh / n * y
    out_ref[...] = t


def _seg_sum_kernel(taug, ei_h, zeros_h, zc_h, out, out_cnt,
                    rows, srcb, dst_all, cnt_t, acc_sh, semg, sems, semi):
    cid = lax.axis_index("c")
    sid = lax.axis_index("s")
    tid = cid * NS + sid

    pltpu.sync_copy(zeros_h, acc_sh.at[pl.ds(sid * RPT, RPT)])
    pltpu.sync_copy(ei_h.at[1, tid], dst_all)
    pltpu.sync_copy(zc_h, cnt_t)
    plsc.subcore_barrier()

    def fire_gather(c, b):
        pltpu.async_copy(taug.at[srcb[b]], rows[b], semg[b])

    def wait_gather(b):
        pltpu.make_async_copy(taug.at[srcb[b]], rows[b], semg[b]).wait()

    ones16 = jnp.ones((16,), jnp.float32)
    tailmask = lax.iota(jnp.int32, 16) >= 8

    def count_chunk(c):
        v0 = dst_all[c, pl.ds(0, 16)]
        plsc.addupdate_scatter(cnt_t, [v0], ones16)
        v1 = dst_all[c, pl.ds(16, 16)]
        plsc.addupdate_scatter(cnt_t, [v1], ones16)
        v2 = dst_all[c, pl.ds(24, 16)]
        plsc.addupdate_scatter(cnt_t, [v2], ones16, mask=tailmask)

    def fire_scatter(c, b):
        pltpu.async_copy(rows[b], acc_sh.at[dst_all.at[c]], sems[b], add=True)

    def wait_scatter(c, b):
        pltpu.make_async_copy(rows[b], acc_sh.at[dst_all.at[c]],
                              sems[b]).wait()

    def fire_idx(c, b):
        pltpu.async_copy(ei_h.at[0, tid, c], srcb[b], semi[b])

    def wait_idx(b):
        pltpu.make_async_copy(ei_h.at[0, tid, 0], srcb[b], semi[b]).wait()

    for b in range(S):
        pltpu.sync_copy(ei_h.at[0, tid, b], srcb[b])
    for b in range(GLEAD):
        fire_gather(b, b)

    def turn(c, b, *, sync_idx=False, no_idx=False, no_gather=False):
        count_chunk(c)
        if not no_gather:
            b2 = (b + GLEAD) % S
            if not sync_idx:
                wait_scatter(c - (S - GLEAD), b2)
                wait_idx(b2)
            fire_gather(c + GLEAD, b2)
        wait_gather(b)
        fire_scatter(c, b)
        if not no_idx:
            fire_idx(c + S, b)

    for b in range(S):
        turn(b, b, sync_idx=(b < S - GLEAD))

    def body(g, carry):
        for b in range(S):
            c = g * S + b
            turn(c, b)
        return carry

    lax.fori_loop(1, NCHUNK // S - 1, body, 0)

    for b in range(S):
        turn(NCHUNK - S + b, b, no_idx=True, no_gather=(b >= S - GLEAD))
    for b in range(S):
        wait_scatter(NCHUNK - S + b, b)

    pltpu.sync_copy(cnt_t, out_cnt.at[tid])
    plsc.subcore_barrier()
    pltpu.sync_copy(acc_sh.at[pl.ds(sid * RPT, RPT)],
                    out.at[cid, pl.ds(sid * RPT, RPT)])


def _finish_kernel(part_ref, cnt_ref, w_ref, c_ref, out_ref):
    p = part_ref[...]
    agg = p[0] + p[1]
    c = c_ref[0, 0]
    rc = jnp.sqrt(c)
    cnt = jnp.clip(jnp.sum(cnt_ref[...], axis=1, keepdims=True), 1.0)
    neigh = jnp.dot(agg, w_ref[...], preferred_element_type=jnp.float32) / cnt
    m = jnp.sqrt(jnp.sum(neigh * neigh, axis=1, keepdims=True))
    m = jnp.clip(m, 1e-10)
    out_ref[...] = jnp.tanh(rc * m * 0.5) * neigh / (rc * m)


def kernel(edge_index, node_embeddings, gc_weight, curvature):
    c2d = curvature.reshape(1, 1).astype(jnp.float32)
    ei = edge_index.astype(jnp.int32).reshape(2, NW, NCHUNK, K)

    rows_blk = 2000
    taug = pl.pallas_call(
        _scale_kernel,
        grid=(N // rows_blk,),
        in_specs=[
            pl.BlockSpec((rows_blk, D), lambda i: (i, 0)),
            pl.BlockSpec(memory_space=pltpu.SMEM),
        ],
        out_specs=pl.BlockSpec((rows_blk, D), lambda i: (i, 0)),
        out_shape=jax.ShapeDtypeStruct((N, D), jnp.float32),
    )(node_embeddings, c2d)

    zeros_h = jnp.zeros((RPT, D), dtype=jnp.float32)
    zc_h = jnp.zeros((N,), dtype=jnp.float32)

    mesh = plsc.VectorSubcoreMesh(core_axis_name="c", subcore_axis_name="s",
                                  num_cores=NC, num_subcores=NS)
    partials, counts = pl.kernel(
        _seg_sum_kernel,
        out_type=[jax.ShapeDtypeStruct((NC, N, D), jnp.float32),
                  jax.ShapeDtypeStruct((NW, N), jnp.float32)],
        mesh=mesh,
        scratch_types=[
            [pltpu.VMEM((K, D), jnp.float32) for _ in range(S)],
            [pltpu.VMEM((K,), jnp.int32) for _ in range(S)],
            pltpu.VMEM((NCHUNK, K), jnp.int32),
            pltpu.VMEM((N,), jnp.float32),
            pltpu.VMEM_SHARED((N, D), jnp.float32),
            [pltpu.SemaphoreType.DMA for _ in range(S)],
            [pltpu.SemaphoreType.DMA for _ in range(S)],
            [pltpu.SemaphoreType.DMA for _ in range(S)],
        ],
        compiler_params=pltpu.CompilerParams(use_tc_tiling_on_sc=False,
                                             needs_layout_passes=False),
    )(taug, ei, zeros_h, zc_h)

    out = pl.pallas_call(
        _finish_kernel,
        grid=(N // rows_blk,),
        in_specs=[
            pl.BlockSpec((NC, rows_blk, D), lambda i: (0, i, 0)),
            pl.BlockSpec((rows_blk, NW), lambda i: (i, 0)),
            pl.BlockSpec((D, D), lambda i: (0, 0)),
            pl.BlockSpec(memory_space=pltpu.SMEM),
        ],
        out_specs=pl.BlockSpec((rows_blk, D), lambda i: (i, 0)),
        out_shape=jax.ShapeDtypeStruct((N, D), jnp.float32),
    )(partials, counts.T, gc_weight, c2d)

    return out
